# baseline (speedup 1.0000x reference)
import functools

import jax
import jax.numpy as jnp
from jax import lax
from jax.experimental import pallas as pl
from jax.experimental.pallas import tpu as pltpu
from jax.experimental.pallas import tpu_sc as plsc

_E = 16
_TOPK = 2
_ALPHA = 0.01

_NW = 32
_LANES = 16
_BLK_ROWS = 1024
_CHUNKS = 1


def _matmul_body(rows_per_w, w_ref, xa_ref, xb_ref, out_ref):
    hh = xa_ref.shape[1]
    res = jax.lax.dot_general(
        w_ref[:, :hh], xa_ref[...],
        dimension_numbers=(((1,), (1,)), ((), ())),
        preferred_element_type=jnp.float32,
    ) + jax.lax.dot_general(
        w_ref[:, hh:], xb_ref[...],
        dimension_numbers=(((1,), (1,)), ((), ())),
        preferred_element_type=jnp.float32,
    )
    w_per_blk = out_ref.shape[0]
    for j in range(w_per_blk):
        out_ref[j] = res[:, j * rows_per_w:(j + 1) * rows_per_w]


def _routing_body(rows_per_w, n, lg_hbm, scores_hbm, idx_hbm, counts_hbm,
                  lg_v, s1_v, s2_v, i1_v, i2_v, cnt_v):
    cid = lax.axis_index("c")
    sid = lax.axis_index("s")
    wid = sid * 2 + cid
    base = wid * rows_per_w

    pltpu.sync_copy(lg_hbm.at[wid], lg_v)

    lanes = lax.iota(jnp.int32, _LANES)
    neg_inf = jnp.full((_LANES,), -jnp.inf, jnp.float32)
    zeros_i = jnp.zeros((_LANES,), jnp.int32)
    zeros_f = jnp.zeros((_LANES,), jnp.float32)
    groups = rows_per_w // _LANES

    def group(g, acc):
        m1, m2 = neg_inf, neg_inf
        i1, i2 = zeros_i, zeros_i
        for e in range(_E):
            c = lg_v[e, pl.ds(g * _LANES, _LANES)]
            es = jnp.full((_LANES,), e, jnp.int32)
            gt1 = c > m1
            gt2 = c > m2
            m2 = jnp.where(gt1, m1, jnp.where(gt2, c, m2))
            i2 = jnp.where(gt1, i1, jnp.where(gt2, es, i2))
            m1 = jnp.where(gt1, c, m1)
            i1 = jnp.where(gt1, es, i1)
        t = jnp.exp(m2 - m1)
        denom = 1.0 + t
        p1 = 1.0 / denom
        p2 = t / denom
        sl = pl.ds(g * _LANES, _LANES)
        s1_v[sl] = p1
        s2_v[sl] = p2
        i1_v[sl] = i1
        i2_v[sl] = i2
        return tuple(
            acc[e]
            + jnp.where(i1 == e, 1.0, 0.0)
            + jnp.where(i2 == e, 1.0, 0.0)
            for e in range(_E)
        )

    acc = lax.fori_loop(0, groups, group, (zeros_f,) * _E)
    for e in range(_E):
        cnt_v[e] = acc[e]

    pltpu.sync_copy(s1_v, scores_hbm.at[pl.ds(base, rows_per_w)])
    pltpu.sync_copy(s2_v, scores_hbm.at[pl.ds(n + base, rows_per_w)])
    pltpu.sync_copy(i1_v, idx_hbm.at[pl.ds(base, rows_per_w)])
    pltpu.sync_copy(i2_v, idx_hbm.at[pl.ds(n + base, rows_per_w)])
    pltpu.sync_copy(cnt_v, counts_hbm.at[wid])


def _loss_body(n_tokens, cnt_ref, out_ref):
    counts = jnp.sum(cnt_ref[...], axis=(0, 1, 3))
    load = counts * (1.0 / n_tokens)
    d = load - (1.0 / _E)
    out_ref[0, 0] = _ALPHA * jnp.sum(d * d)


def kernel(x, W):
    bsz, seq, h = x.shape
    n = bsz * seq
    x_flat = x.reshape(n, h)

    nc = n // _CHUNKS
    rows_per_w = nc // _NW
    blk = min(_BLK_ROWS, nc)
    w_per_blk = blk // rows_per_w

    mesh = plsc.VectorSubcoreMesh(
        core_axis_name="c", subcore_axis_name="s",
        num_cores=2, num_subcores=16)
    route = pl.kernel(
        functools.partial(_routing_body, rows_per_w, nc),
        out_type=[
            jax.ShapeDtypeStruct((nc * 2,), jnp.float32),
            jax.ShapeDtypeStruct((nc * 2,), jnp.int32),
            jax.ShapeDtypeStruct((_NW, _E, _LANES), jnp.float32),
        ],
        mesh=mesh,
        scratch_types=[
            pltpu.VMEM((_E, rows_per_w), jnp.float32),
            pltpu.VMEM((rows_per_w,), jnp.float32),
            pltpu.VMEM((rows_per_w,), jnp.float32),
            pltpu.VMEM((rows_per_w,), jnp.int32),
            pltpu.VMEM((rows_per_w,), jnp.int32),
            pltpu.VMEM((_E, _LANES), jnp.float32),
        ],
    )

    def make_mm(c):
        off = c * (nc // blk)
        return pl.pallas_call(
            functools.partial(_matmul_body, rows_per_w),
            grid=(nc // blk,),
            in_specs=[
                pl.BlockSpec((_E, h), lambda i: (0, 0)),
                pl.BlockSpec((blk, h // 2), lambda i, _o=off: (_o + i, 0)),
                pl.BlockSpec((blk, h // 2), lambda i, _o=off: (_o + i, 1)),
            ],
            out_specs=pl.BlockSpec((w_per_blk, _E, rows_per_w),
                                   lambda i: (i, 0, 0)),
            out_shape=jax.ShapeDtypeStruct((_NW, _E, rows_per_w),
                                           jnp.float32),
        )

    s_parts, i_parts, c_parts = [], [], []
    for c in range(_CHUNKS):
        logits = make_mm(c)(W, x_flat, x_flat)
        s_c, i_c, p_c = route(logits)
        s_parts.append(s_c.reshape(2, nc))
        i_parts.append(i_c.reshape(2, nc))
        c_parts.append(p_c)

    pcounts = jnp.stack(c_parts)
    loss = pl.pallas_call(
        functools.partial(_loss_body, n),
        out_shape=jax.ShapeDtypeStruct((1, 1), jnp.float32),
        out_specs=pl.BlockSpec(memory_space=pltpu.SMEM),
    )(pcounts)

    scores = jnp.concatenate(s_parts, axis=1).T
    idx = jnp.concatenate(i_parts, axis=1).T
    return scores, idx, loss[0, 0]


# speedup vs baseline: 1.0287x; 1.0287x over previous
import functools

import jax
import jax.numpy as jnp
from jax import lax
from jax.experimental import pallas as pl
from jax.experimental.pallas import tpu as pltpu
from jax.experimental.pallas import tpu_sc as plsc

_E = 16
_ALPHA = 0.01
_NW = 32
_LANES = 16
_BLK_ROWS = 1024


def _matmul_body(rows_per_w, w_ref, x_ref, out_ref):
    res = jax.lax.dot_general(
        w_ref[...], x_ref[...],
        dimension_numbers=(((1,), (1,)), ((), ())),
        preferred_element_type=jnp.float32,
    )
    w_per_blk = out_ref.shape[0]
    for j in range(w_per_blk):
        out_ref[j] = res[:, j * rows_per_w:(j + 1) * rows_per_w]


def _routing_body(rows_per_w, n, lg_hbm, scores_hbm, idx_hbm, counts_hbm,
                  lg_v, s1_v, s2_v, i1_v, i2_v, cnt_v):
    cid = lax.axis_index("c")
    sid = lax.axis_index("s")
    wid = sid * 2 + cid
    base = wid * rows_per_w

    pltpu.sync_copy(lg_hbm.at[wid], lg_v)

    neg_inf = jnp.full((_LANES,), -jnp.inf, jnp.float32)
    zeros_i = jnp.zeros((_LANES,), jnp.int32)
    zeros_f = jnp.zeros((_LANES,), jnp.float32)
    groups = rows_per_w // _LANES

    def group(g, acc):
        m1, m2 = neg_inf, neg_inf
        i1, i2 = zeros_i, zeros_i
        for e in range(_E):
            c = lg_v[e, pl.ds(g * _LANES, _LANES)]
            es = jnp.full((_LANES,), e, jnp.int32)
            gt1 = c > m1
            gt2 = c > m2
            m2 = jnp.where(gt1, m1, jnp.where(gt2, c, m2))
            i2 = jnp.where(gt1, i1, jnp.where(gt2, es, i2))
            m1 = jnp.where(gt1, c, m1)
            i1 = jnp.where(gt1, es, i1)
        t = jnp.exp(m2 - m1)
        denom = 1.0 + t
        p1 = 1.0 / denom
        p2 = t / denom
        sl = pl.ds(g * _LANES, _LANES)
        s1_v[sl] = p1
        s2_v[sl] = p2
        i1_v[sl] = i1
        i2_v[sl] = i2
        return tuple(
            acc[e]
            + jnp.where(i1 == e, 1.0, 0.0)
            + jnp.where(i2 == e, 1.0, 0.0)
            for e in range(_E)
        )

    acc = lax.fori_loop(0, groups, group, (zeros_f,) * _E)
    for e in range(_E):
        cnt_v[e] = acc[e]

    pltpu.sync_copy(s1_v, scores_hbm.at[pl.ds(base, rows_per_w)])
    pltpu.sync_copy(s2_v, scores_hbm.at[pl.ds(n + base, rows_per_w)])
    pltpu.sync_copy(i1_v, idx_hbm.at[pl.ds(base, rows_per_w)])
    pltpu.sync_copy(i2_v, idx_hbm.at[pl.ds(n + base, rows_per_w)])
    pltpu.sync_copy(cnt_v, counts_hbm.at[wid])


def _loss_body(n_tokens, cnt_ref, out_ref):
    counts = jnp.sum(cnt_ref[...], axis=(0, 2))
    load = counts * (1.0 / n_tokens)
    d = load - (1.0 / _E)
    out_ref[0, 0] = _ALPHA * jnp.sum(d * d)


def kernel(x, W):
    bsz, seq, h = x.shape
    n = bsz * seq
    rows_per_w = n // _NW
    x_flat = x.reshape(n, h)

    blk = _BLK_ROWS
    w_per_blk = blk // rows_per_w
    logits = pl.pallas_call(
        functools.partial(_matmul_body, rows_per_w),
        grid=(n // blk,),
        in_specs=[
            pl.BlockSpec((_E, h), lambda i: (0, 0)),
            pl.BlockSpec((blk, h), lambda i: (i, 0)),
        ],
        out_specs=pl.BlockSpec((w_per_blk, _E, rows_per_w),
                               lambda i: (i, 0, 0)),
        out_shape=jax.ShapeDtypeStruct((_NW, _E, rows_per_w), jnp.float32),
    )(W, x_flat)

    mesh = plsc.VectorSubcoreMesh(
        core_axis_name="c", subcore_axis_name="s",
        num_cores=2, num_subcores=16)
    route = pl.kernel(
        functools.partial(_routing_body, rows_per_w, n),
        out_type=[
            jax.ShapeDtypeStruct((n * 2,), jnp.float32),
            jax.ShapeDtypeStruct((n * 2,), jnp.int32),
            jax.ShapeDtypeStruct((_NW, _E, _LANES), jnp.float32),
        ],
        mesh=mesh,
        scratch_types=[
            pltpu.VMEM((_E, rows_per_w), jnp.float32),
            pltpu.VMEM((rows_per_w,), jnp.float32),
            pltpu.VMEM((rows_per_w,), jnp.float32),
            pltpu.VMEM((rows_per_w,), jnp.int32),
            pltpu.VMEM((rows_per_w,), jnp.int32),
            pltpu.VMEM((_E, _LANES), jnp.float32),
        ],
    )
    scores_flat, idx_flat, pcounts = route(logits)

    loss = pl.pallas_call(
        functools.partial(_loss_body, n),
        out_shape=jax.ShapeDtypeStruct((1, 1), jnp.float32),
        out_specs=pl.BlockSpec(memory_space=pltpu.SMEM),
    )(pcounts)

    scores = scores_flat.reshape(2, n).T
    idx = idx_flat.reshape(2, n).T
    return scores, idx, loss[0, 0]
